# baseline (speedup 1.0000x reference)
import functools

import jax
import jax.numpy as jnp
from jax import lax
from jax.experimental import pallas as pl
from jax.experimental.pallas import tpu as pltpu
from jax.experimental.pallas import tpu_sc as plsc

VOCAB = 100000
D = 64
S = 200
B = 1024
NW = 32
LANES = 16
GB = 128
SB = 8
NBC = 4
UNITS = (S // SB) * (B // GB)


def _make_kernel():
    mesh = plsc.VectorSubcoreMesh(core_axis_name="c", subcore_axis_name="s")

    @functools.partial(
        pl.kernel,
        mesh=mesh,
        out_type=jax.ShapeDtypeStruct((S, D // 8, B // GB, 8, GB), jnp.float32),
        scratch_types=[
            pltpu.VMEM((4, GB), jnp.int32),
            pltpu.VMEM((NBC, GB, D), jnp.float32),
            pltpu.VMEM((2, D // 8, 8, GB), jnp.float32),
            pltpu.VMEM((S * D,), jnp.float32),
            pltpu.SemaphoreType.DMA((4,)),
            pltpu.SemaphoreType.DMA((NBC,)),
            pltpu.SemaphoreType.DMA((2,)),
        ],
        compiler_params=pltpu.CompilerParams(use_tc_tiling_on_sc=False,
                                             needs_layout_passes=False),
    )
    def k(seq_hbm, tok_hbm, pos_hbm, out_hbm,
          idx_v, rows_v, stage_v, pos_v, isem, gsem, osem):
        wid = lax.axis_index("s") * 2 + lax.axis_index("c")
        units_w = jnp.where(wid < UNITS % NW, UNITS // NW + 1, UNITS // NW)
        nchunks = units_w * SB

        pltpu.sync_copy(pos_hbm, pos_v)

        def coords(c):
            u = wid + ((c >> 3) << 5)
            sh = u >> 3
            bh = u & 7
            s = (sh << 3) + (c & 7)
            return bh, s

        def idx_desc(c, ib):
            bh, s = coords(c)
            return pltpu.make_async_copy(
                seq_hbm.at[s >> 3, bh, s & 7], idx_v.at[ib], isem.at[ib])

        def gather_desc(c, ib, gb):
            return pltpu.make_async_copy(
                tok_hbm.at[idx_v.at[ib]], rows_v.at[gb], gsem.at[gb])

        def out_desc(c, sp):
            bh, s = coords(c)
            return pltpu.make_async_copy(
                stage_v.at[sp], out_hbm.at[s, :, bh], osem.at[sp])

        iota16 = lax.iota(jnp.int32, LANES)

        def transpose_add(c, gb, sp):
            _, s = coords(c)
            pbase = s * D

            def dh_body(dh, carry):
                for dl in range(8):
                    d = dh * 8 + dl
                    psp = plsc.load_gather(
                        pos_v, [jnp.full((LANES,), pbase + d, jnp.int32)])
                    for bc in range(GB // LANES):
                        col = plsc.load_gather(
                            rows_v.at[gb],
                            [bc * LANES + iota16,
                             jnp.full((LANES,), d, jnp.int32)])
                        stage_v[sp, dh, dl, pl.ds(bc * LANES, LANES)] = (
                            col + psp)
                return carry
            lax.fori_loop(0, D // 8, dh_body, 0)

        for c0 in range(3):
            idx_desc(c0, c0).start()
        for c0 in range(2):
            idx_desc(c0, c0).wait()
            gather_desc(c0, c0, c0).start()

        def body(i, carry):
            cb = i * 4
            for b in range(4):
                c = cb + b

                @pl.when(c + 3 < nchunks)
                def _pf_idx():
                    idx_desc(c + 3, (b + 3) % 4).start()

                @pl.when(c + 2 < nchunks)
                def _pf_gather():
                    idx_desc(c + 2, (b + 2) % 4).wait()
                    gather_desc(c + 2, (b + 2) % 4, (b + 2) % NBC).start()

                @pl.when(c >= 2)
                def _drain_out():
                    out_desc(c - 2, b % 2).wait()

                gather_desc(c, b, b % NBC).wait()
                transpose_add(c, b % NBC, b % 2)
                out_desc(c, b % 2).start()
            return carry

        lax.fori_loop(0, nchunks >> 2, body, 0)

        out_desc(nchunks - 2, 0).wait()
        out_desc(nchunks - 1, 1).wait()

    return k


_kernel_call = _make_kernel()


def kernel(sequence, token_table, pos_table):
    seq_p = (sequence.astype(jnp.int32).transpose(1, 0)
             .reshape(S // SB, SB, B // GB, GB).transpose(0, 2, 1, 3))
    pos_flat = pos_table.reshape(-1)
    out_p = _kernel_call(seq_p, token_table, pos_flat)
    return out_p.transpose(2, 4, 0, 1, 3).reshape(B, S, D)


# speedup vs baseline: 2.7629x; 2.7629x over previous
import functools

import jax
import jax.numpy as jnp
from jax import lax
from jax.experimental import pallas as pl
from jax.experimental.pallas import tpu as pltpu
from jax.experimental.pallas import tpu_sc as plsc

VOCAB = 100000
D = 64
DP = 128
S = 200
B = 1024
NW = 32
SEQ_PER_W = B // NW
LANES = 16
NB = 4
SPLIT = 128


def _make_kernel():
    mesh = plsc.VectorSubcoreMesh(core_axis_name="c", subcore_axis_name="s")

    @functools.partial(
        pl.kernel,
        mesh=mesh,
        out_type=jax.ShapeDtypeStruct((B * S, DP), jnp.float32),
        scratch_types=[
            pltpu.VMEM((SEQ_PER_W * S,), jnp.int32),
            pltpu.VMEM((NB, S, D), jnp.float32),
            pltpu.VMEM((S * D,), jnp.float32),
            pltpu.SemaphoreType.DMA((NB,)),
            pltpu.SemaphoreType.DMA((NB,)),
        ],
        compiler_params=pltpu.CompilerParams(use_tc_tiling_on_sc=False),
    )
    def k(seq_hbm, tok_hbm, pos_hbm, out_hbm, idx_v, rows_v, pos_v, gsem, osem):
        wid = lax.axis_index("s") * 2 + lax.axis_index("c")
        base_row = wid * (SEQ_PER_W * S)

        pltpu.sync_copy(seq_hbm.at[pl.ds(base_row, SEQ_PER_W * S)], idx_v)
        pltpu.sync_copy(pos_hbm, pos_v)

        def gather_descs(s, b):
            off = s * S
            c1 = pltpu.make_async_copy(
                tok_hbm.at[idx_v.at[pl.ds(off, SPLIT)]],
                rows_v.at[b, pl.ds(0, SPLIT)], gsem.at[b])
            c2 = pltpu.make_async_copy(
                tok_hbm.at[idx_v.at[pl.ds(off + SPLIT, S - SPLIT)]],
                rows_v.at[b, pl.ds(SPLIT, S - SPLIT)], gsem.at[b])
            return c1, c2

        def out_desc(s, b):
            return pltpu.make_async_copy(
                rows_v.at[b],
                out_hbm.at[pl.ds(base_row + s * S, S), pl.ds(0, D)],
                osem.at[b])

        def add_slot(b):
            def add_body(i, c):
                r = i * 4
                for j in range(4):
                    for q in range(D // LANES):
                        plsc.addupdate(
                            rows_v.at[b, r + j, pl.ds(q * LANES, LANES)],
                            pos_v[pl.ds((r + j) * D + q * LANES, LANES)],
                        )
                return c
            lax.fori_loop(0, S // 4, add_body, 0)

        for s0 in range(NB - 1):
            c1, c2 = gather_descs(s0, s0)
            c1.start()
            c2.start()

        def body(i, carry):
            s_base = i * NB
            for b in range(NB):
                s = s_base + b
                c1, c2 = gather_descs(s, b)
                c1.wait()
                c2.wait()
                add_slot(b)
                out_desc(s, b).start()
                t = s + NB - 1
                bt = (b + NB - 1) % NB

                @pl.when(t < SEQ_PER_W)
                def _prefetch():
                    @pl.when(s >= 1)
                    def _drain_prev():
                        out_desc(s - 1, bt).wait()
                    g1, g2 = gather_descs(t, bt)
                    g1.start()
                    g2.start()
            return carry

        lax.fori_loop(0, SEQ_PER_W // NB, body, 0)

        for b in range(NB):
            out_desc(SEQ_PER_W - NB + b, b).wait()

    return k


_kernel_call = _make_kernel()


def kernel(sequence, token_table, pos_table):
    seq_flat = sequence.reshape(-1).astype(jnp.int32)
    pos_flat = pos_table.reshape(-1)
    out = _kernel_call(seq_flat, token_table, pos_flat)
    return out[:, :D].reshape(B, S, D)
